# Initial kernel scaffold; baseline (speedup 1.0000x reference)
import functools

import jax
import jax.numpy as jnp
import numpy as np
from jax.experimental import pallas as pl
from jax.experimental.pallas import tpu as pltpu

_D = 512
_M = 128
_L = 100
_N = 256
_TOPK = 8
_KWIN = 25
_NB = 8
_TOK = _NB * _L
_T = _N * _L


def _avg_matrix():
    pad = (_KWIN - 1) // 2
    src = np.clip(np.arange(_L + 2 * pad) - pad, 0, _L - 1)
    m = np.zeros((_L, _L), np.float32)
    for i in range(_L):
        for j in src[i:i + _KWIN]:
            m[i, j] += 1.0 / _KWIN
    return m


_BD = np.kron(np.eye(_NB, dtype=np.float32), _avg_matrix())


def _fused_body(x_ref, bd_ref, w_in_ref, b_in_ref, w_tr_ref, b_tr_ref,
                mem_ref, mem_t_ref, invn_ref, msq_ref, wsig_ref, bsig_ref,
                w_dec_ref, b_dec_ref,
                out_ref, season_ref, sigma_ref, loss_ref):
    f32 = jnp.float32
    xb = x_ref[...]
    w_in = w_in_ref[...]
    h = jnp.dot(xb, w_in, preferred_element_type=f32) + b_in_ref[...]
    xavg = jnp.dot(bd_ref[...], xb, preferred_element_type=f32)
    trend = jnp.dot(xavg, w_in, preferred_element_type=f32) + b_in_ref[...]
    season = h - trend
    season_ref[...] = season

    norm = jnp.sqrt(jnp.sum(season * season, axis=1, keepdims=True))
    q = season / (norm + 1e-12)

    trend_out = jnp.dot(trend, w_tr_ref[...], preferred_element_type=f32)
    trend_out = trend_out + b_tr_ref[...]

    att_raw = jnp.dot(q, mem_t_ref[...], preferred_element_type=f32)
    att = att_raw * 10.0

    lane = jax.lax.broadcasted_iota(jnp.int32, (_TOK, _M), 1)

    cur = att
    topmask = jnp.zeros((_TOK, _M), dtype=jnp.bool_)
    for _ in range(_TOPK):
        mval = jnp.max(cur, axis=1, keepdims=True)
        idx = jnp.min(jnp.where(cur == mval, lane, _M), axis=1, keepdims=True)
        sel = lane == idx
        topmask = jnp.logical_or(topmask, sel)
        cur = jnp.where(sel, -jnp.inf, cur)

    amax = jnp.max(att, axis=1, keepdims=True)
    p = jnp.where(topmask, jnp.exp(att - amax), 0.0)
    p = p / jnp.sum(p, axis=1, keepdims=True)
    read = jnp.dot(p, mem_ref[...], preferred_element_type=f32)

    w_dec = w_dec_ref[...]
    out = jnp.dot(h, w_dec[0:_D], preferred_element_type=f32)
    out += jnp.dot(trend_out, w_dec[_D:2 * _D], preferred_element_type=f32)
    out += jnp.dot(read, w_dec[2 * _D:3 * _D], preferred_element_type=f32)
    out_ref[...] = out + b_dec_ref[...]

    sig = jnp.sum(q * wsig_ref[...], axis=1, keepdims=True) + bsig_ref[0, 0]
    sigma_ref[...] = sig

    sim = att_raw * invn_ref[...]
    smax = jnp.max(sim, axis=1, keepdims=True)
    lse = jnp.log(jnp.sum(jnp.exp(sim - smax), axis=1, keepdims=True)) + smax
    closs = jnp.sum(lse - smax) * (1.0 / _T)

    lidx = jnp.min(jnp.where(sim == smax, lane, _M), axis=1, keepdims=True)
    lsel = lane == lidx
    att_sel = jnp.sum(jnp.where(lsel, att_raw, 0.0), axis=1, keepdims=True)
    msq_sel = jnp.sum(jnp.where(lsel, msq_ref[...], 0.0), axis=1, keepdims=True)
    qsq = jnp.sum(q * q, axis=1, keepdims=True)
    gloss = jnp.sum(qsq - 2.0 * att_sel + msq_sel) * (1.0 / (_T * _D))

    sig2 = sig * sig + 1e-6
    kld = jnp.sum(sig2 - jnp.log(sig2) - 1.0) * (0.5 / _T)

    li = jax.lax.broadcasted_iota(jnp.int32, (1, _M), 1)
    vals = (jnp.where(li == 0, closs, 0.0) + jnp.where(li == 1, gloss, 0.0)
            + jnp.where(li == 2, kld, 0.0))

    @pl.when(pl.program_id(0) == 0)
    def _():
        loss_ref[...] = jnp.zeros_like(loss_ref)

    loss_ref[...] += vals


@functools.partial(jax.jit)
def kernel(x, W_in, b_in, W_tr, b_tr, mem, W_sig, b_sig, W_dec, b_dec):
    x2 = x.reshape(_T, -1)
    enc_in = x2.shape[1]
    bd = jnp.asarray(_BD)
    mem_t = mem.T
    mnorm = jnp.sqrt(jnp.sum(mem * mem, axis=1))
    invn = (10.0 / (mnorm + 1e-12)).reshape(1, _M)
    msq = (mnorm * mnorm).reshape(1, _M)
    wsig = W_sig.reshape(1, _D)
    bsig = jnp.broadcast_to(b_sig.reshape(1, 1), (1, 8))
    b_in2 = b_in.reshape(1, _D)
    b_tr2 = b_tr.reshape(1, _D)
    b_dec2 = b_dec.reshape(1, -1)

    grid = (_T // _TOK,)
    tok_spec = lambda cols: pl.BlockSpec((_TOK, cols), lambda i: (i, 0))
    full = lambda shape: pl.BlockSpec(shape, lambda i: (0,) * len(shape))

    out2, season2, sigma2, losses = pl.pallas_call(
        _fused_body,
        grid=grid,
        in_specs=[
            tok_spec(enc_in),
            full((_TOK, _TOK)),
            full((enc_in, _D)),
            full((1, _D)),
            full((_D, _D)),
            full((1, _D)),
            full((_M, _D)),
            full((_D, _M)),
            full((1, _M)),
            full((1, _M)),
            full((1, _D)),
            full((1, 8)),
            full((3 * _D, enc_in)),
            full((1, enc_in)),
        ],
        out_specs=[
            tok_spec(enc_in),
            tok_spec(_D),
            pl.BlockSpec((_TOK, 1), lambda i: (i, 0)),
            pl.BlockSpec((1, _M), lambda i: (0, 0)),
        ],
        out_shape=[
            jax.ShapeDtypeStruct((_T, enc_in), jnp.float32),
            jax.ShapeDtypeStruct((_T, _D), jnp.float32),
            jax.ShapeDtypeStruct((_T, 1), jnp.float32),
            jax.ShapeDtypeStruct((1, _M), jnp.float32),
        ],
        compiler_params=pltpu.CompilerParams(
            dimension_semantics=("arbitrary",),
        ),
    )(x2, bd, W_in, b_in2, W_tr, b_tr2, mem, mem_t, invn, msq, wsig, bsig,
      W_dec, b_dec2)

    out = out2.reshape(_N, _L, enc_in)
    season = season2.reshape(_N, _L, _D)
    sigma = sigma2.reshape(_N, _L)
    return (out, mem, season, losses[0, 0], losses[0, 1], losses[0, 2], sigma)


# speedup vs baseline: 6.7467x; 6.7467x over previous
import functools

import jax
import jax.numpy as jnp
import numpy as np
from jax.experimental import pallas as pl
from jax.experimental.pallas import tpu as pltpu

_D = 512
_M = 128
_L = 100
_N = 256
_TOPK = 8
_KWIN = 25
_NB = 8
_TOK = _NB * _L
_T = _N * _L


def _avg_matrix():
    pad = (_KWIN - 1) // 2
    src = np.clip(np.arange(_L + 2 * pad) - pad, 0, _L - 1)
    m = np.zeros((_L, _L), np.float32)
    for i in range(_L):
        for j in src[i:i + _KWIN]:
            m[i, j] += 1.0 / _KWIN
    return m


_BD = np.kron(np.eye(_NB, dtype=np.float32), _avg_matrix())


def _fused_body(x_ref, bd_ref, w_in_ref, b_in_ref, w_tr_ref, b_tr_ref,
                mem_ref, mem_t_ref, invn_ref, msq_ref, wsig_ref, bsig_ref,
                w_dec_ref, b_dec_ref,
                out_ref, season_ref, sigma_ref, loss_ref):
    f32 = jnp.float32
    xb = x_ref[...]
    w_in = w_in_ref[...]
    h = jnp.dot(xb, w_in, preferred_element_type=f32) + b_in_ref[...]
    xavg = jnp.dot(bd_ref[...], xb, preferred_element_type=f32)
    trend = jnp.dot(xavg, w_in, preferred_element_type=f32) + b_in_ref[...]
    season = h - trend
    norm = jnp.sqrt(jnp.sum(season * season, axis=1, keepdims=True))
    q = season / (norm + 1e-12)
    season_ref[...] = q

    trend_out = jnp.dot(trend, w_tr_ref[...], preferred_element_type=f32)
    trend_out = trend_out + b_tr_ref[...]

    att_raw = jnp.dot(q, mem_t_ref[...], preferred_element_type=f32)
    att = att_raw * 10.0

    lane = jax.lax.broadcasted_iota(jnp.int32, (_TOK, _M), 1)

    cur = att
    topmask = jnp.zeros((_TOK, _M), dtype=jnp.bool_)
    for _ in range(_TOPK):
        mval = jnp.max(cur, axis=1, keepdims=True)
        idx = jnp.min(jnp.where(cur == mval, lane, _M), axis=1, keepdims=True)
        sel = lane == idx
        topmask = jnp.logical_or(topmask, sel)
        cur = jnp.where(sel, -jnp.inf, cur)

    amax = jnp.max(att, axis=1, keepdims=True)
    p = jnp.where(topmask, jnp.exp(att - amax), 0.0)
    p = p / jnp.sum(p, axis=1, keepdims=True)
    read = jnp.dot(p, mem_ref[...], preferred_element_type=f32)

    w_dec = w_dec_ref[...]
    out = jnp.dot(h, w_dec[0:_D], preferred_element_type=f32)
    out += jnp.dot(trend_out, w_dec[_D:2 * _D], preferred_element_type=f32)
    out += jnp.dot(read, w_dec[2 * _D:3 * _D], preferred_element_type=f32)
    out_ref[...] = out + b_dec_ref[...]

    sig = jnp.sum(q * wsig_ref[...], axis=1, keepdims=True) + bsig_ref[0, 0]
    sigma_ref[...] = sig

    sim = att_raw * invn_ref[...]
    smax = jnp.max(sim, axis=1, keepdims=True)
    lse = jnp.log(jnp.sum(jnp.exp(sim - smax), axis=1, keepdims=True)) + smax
    closs = jnp.sum(lse - smax) * (1.0 / _T)

    lidx = jnp.min(jnp.where(sim == smax, lane, _M), axis=1, keepdims=True)
    lsel = lane == lidx
    att_sel = jnp.sum(jnp.where(lsel, att_raw, 0.0), axis=1, keepdims=True)
    msq_sel = jnp.sum(jnp.where(lsel, msq_ref[...], 0.0), axis=1, keepdims=True)
    qsq = jnp.sum(q * q, axis=1, keepdims=True)
    gloss = jnp.sum(qsq - 2.0 * att_sel + msq_sel) * (1.0 / (_T * _D))

    sig2 = sig * sig + 1e-6
    kld = jnp.sum(sig2 - jnp.log(sig2) - 1.0) * (0.5 / _T)

    li = jax.lax.broadcasted_iota(jnp.int32, (1, _M), 1)
    vals = (jnp.where(li == 0, closs, 0.0) + jnp.where(li == 1, gloss, 0.0)
            + jnp.where(li == 2, kld, 0.0))

    @pl.when(pl.program_id(0) == 0)
    def _():
        loss_ref[...] = jnp.zeros_like(loss_ref)

    loss_ref[...] += vals


@functools.partial(jax.jit)
def kernel(x, W_in, b_in, W_tr, b_tr, mem, W_sig, b_sig, W_dec, b_dec):
    x2 = x.reshape(_T, -1)
    enc_in = x2.shape[1]
    bd = jnp.asarray(_BD)
    mem_t = mem.T
    mnorm = jnp.sqrt(jnp.sum(mem * mem, axis=1))
    invn = (10.0 / (mnorm + 1e-12)).reshape(1, _M)
    msq = (mnorm * mnorm).reshape(1, _M)
    wsig = W_sig.reshape(1, _D)
    bsig = jnp.broadcast_to(b_sig.reshape(1, 1), (1, 8))
    b_in2 = b_in.reshape(1, _D)
    b_tr2 = b_tr.reshape(1, _D)
    b_dec2 = b_dec.reshape(1, -1)

    grid = (_T // _TOK,)
    tok_spec = lambda cols: pl.BlockSpec((_TOK, cols), lambda i: (i, 0))
    full = lambda shape: pl.BlockSpec(shape, lambda i: (0,) * len(shape))

    out2, season2, sigma2, losses = pl.pallas_call(
        _fused_body,
        grid=grid,
        in_specs=[
            tok_spec(enc_in),
            full((_TOK, _TOK)),
            full((enc_in, _D)),
            full((1, _D)),
            full((_D, _D)),
            full((1, _D)),
            full((_M, _D)),
            full((_D, _M)),
            full((1, _M)),
            full((1, _M)),
            full((1, _D)),
            full((1, 8)),
            full((3 * _D, enc_in)),
            full((1, enc_in)),
        ],
        out_specs=[
            tok_spec(enc_in),
            tok_spec(_D),
            pl.BlockSpec((_TOK, 1), lambda i: (i, 0)),
            pl.BlockSpec((1, _M), lambda i: (0, 0)),
        ],
        out_shape=[
            jax.ShapeDtypeStruct((_T, enc_in), jnp.float32),
            jax.ShapeDtypeStruct((_T, _D), jnp.float32),
            jax.ShapeDtypeStruct((_T, 1), jnp.float32),
            jax.ShapeDtypeStruct((1, _M), jnp.float32),
        ],
        compiler_params=pltpu.CompilerParams(
            dimension_semantics=("arbitrary",),
        ),
    )(x2, bd, W_in, b_in2, W_tr, b_tr2, mem, mem_t, invn, msq, wsig, bsig,
      W_dec, b_dec2)

    out = out2.reshape(_N, _L, enc_in)
    season = season2.reshape(_N, _L, _D)
    sigma = sigma2.reshape(_N, _L)
    return (out, mem, season, losses[0, 0], losses[0, 1], losses[0, 2], sigma)


# speedup vs baseline: 7.3829x; 1.0943x over previous
import functools

import jax
import jax.numpy as jnp
import numpy as np
from jax.experimental import pallas as pl
from jax.experimental.pallas import tpu as pltpu

_D = 512
_M = 128
_L = 100
_N = 256
_TOPK = 8
_KWIN = 25
_NB = 8
_TOK = _NB * _L
_T = _N * _L


def _avg_matrix():
    pad = (_KWIN - 1) // 2
    src = np.clip(np.arange(_L + 2 * pad) - pad, 0, _L - 1)
    m = np.zeros((_L, _L), np.float32)
    for i in range(_L):
        for j in src[i:i + _KWIN]:
            m[i, j] += 1.0 / _KWIN
    return m


_BD = np.kron(np.eye(_NB, dtype=np.float32), _avg_matrix())


def _fused_body(x_ref, bd_ref, w_in_ref, b_in_ref, w_tr_ref, b_tr_ref,
                mem_ref, mem_t_ref, invn_ref, msq_ref, wsig_ref, bsig_ref,
                w_dec_ref, b_dec_ref,
                out_ref, season_ref, sigma_ref, loss_ref,
                a1_ref, a2_ref, mw_ref, bias_ref):
    f32 = jnp.float32
    dot = functools.partial(jnp.dot, preferred_element_type=f32)
    xb = x_ref[...]
    w_in = w_in_ref[...]

    @pl.when(pl.program_id(0) == 0)
    def _():
        wd = w_dec_ref[...]
        wd1 = wd[0:_D]
        wd2 = wd[_D:2 * _D]
        wd3 = wd[2 * _D:3 * _D]
        w_tr = w_tr_ref[...]
        a1_ref[...] = dot(w_in, wd1)
        a2_ref[...] = dot(dot(w_in, w_tr), wd2)
        mw_ref[...] = dot(mem_ref[...], wd3)
        bias_ref[...] = (dot(b_in_ref[...], wd1)
                         + dot(dot(b_in_ref[...], w_tr) + b_tr_ref[...], wd2)
                         + b_dec_ref[...])
        loss_ref[...] = jnp.zeros_like(loss_ref)

    xavg = dot(bd_ref[...], xb)
    season = dot(xb - xavg, w_in)
    norm = jnp.sqrt(jnp.sum(season * season, axis=1, keepdims=True))
    q = season / (norm + 1e-12)
    season_ref[...] = q

    att_raw = dot(q, mem_t_ref[...])
    att = att_raw * 10.0

    lane = jax.lax.broadcasted_iota(jnp.int32, (_TOK, _M), 1)

    cur = att
    topmask = jnp.zeros((_TOK, _M), dtype=jnp.bool_)
    for _ in range(_TOPK):
        mval = jnp.max(cur, axis=1, keepdims=True)
        idx = jnp.min(jnp.where(cur == mval, lane, _M), axis=1, keepdims=True)
        sel = lane == idx
        topmask = jnp.logical_or(topmask, sel)
        cur = jnp.where(sel, -jnp.inf, cur)

    amax = jnp.max(att, axis=1, keepdims=True)
    p = jnp.where(topmask, jnp.exp(att - amax), 0.0)
    p = p / jnp.sum(p, axis=1, keepdims=True)

    out = dot(xb, a1_ref[...]) + dot(xavg, a2_ref[...]) + dot(p, mw_ref[...])
    out_ref[...] = out + bias_ref[...]

    sig = jnp.sum(q * wsig_ref[...], axis=1, keepdims=True) + bsig_ref[0, 0]
    sigma_ref[...] = sig

    sim = att_raw * invn_ref[...]
    smax = jnp.max(sim, axis=1, keepdims=True)
    lse = jnp.log(jnp.sum(jnp.exp(sim - smax), axis=1, keepdims=True)) + smax
    closs = jnp.sum(lse - smax) * (1.0 / _T)

    lidx = jnp.min(jnp.where(sim == smax, lane, _M), axis=1, keepdims=True)
    lsel = lane == lidx
    att_sel = jnp.sum(jnp.where(lsel, att_raw, 0.0), axis=1, keepdims=True)
    msq_sel = jnp.sum(jnp.where(lsel, msq_ref[...], 0.0), axis=1, keepdims=True)
    qsq = jnp.sum(q * q, axis=1, keepdims=True)
    gloss = jnp.sum(qsq - 2.0 * att_sel + msq_sel) * (1.0 / (_T * _D))

    sig2 = sig * sig + 1e-6
    kld = jnp.sum(sig2 - jnp.log(sig2) - 1.0) * (0.5 / _T)

    li = jax.lax.broadcasted_iota(jnp.int32, (1, _M), 1)
    vals = (jnp.where(li == 0, closs, 0.0) + jnp.where(li == 1, gloss, 0.0)
            + jnp.where(li == 2, kld, 0.0))
    loss_ref[...] += vals


@functools.partial(jax.jit)
def kernel(x, W_in, b_in, W_tr, b_tr, mem, W_sig, b_sig, W_dec, b_dec):
    x2 = x.reshape(_T, -1)
    enc_in = x2.shape[1]
    bd = jnp.asarray(_BD)
    mem_t = mem.T
    mnorm = jnp.sqrt(jnp.sum(mem * mem, axis=1))
    invn = (10.0 / (mnorm + 1e-12)).reshape(1, _M)
    msq = (mnorm * mnorm).reshape(1, _M)
    wsig = W_sig.reshape(1, _D)
    bsig = jnp.broadcast_to(b_sig.reshape(1, 1), (1, 8))
    b_in2 = b_in.reshape(1, _D)
    b_tr2 = b_tr.reshape(1, _D)
    b_dec2 = b_dec.reshape(1, -1)

    grid = (_T // _TOK,)
    tok_spec = lambda cols: pl.BlockSpec((_TOK, cols), lambda i: (i, 0))
    full = lambda shape: pl.BlockSpec(shape, lambda i: (0,) * len(shape))

    out2, season2, sigma2, losses = pl.pallas_call(
        _fused_body,
        grid=grid,
        in_specs=[
            tok_spec(enc_in),
            full((_TOK, _TOK)),
            full((enc_in, _D)),
            full((1, _D)),
            full((_D, _D)),
            full((1, _D)),
            full((_M, _D)),
            full((_D, _M)),
            full((1, _M)),
            full((1, _M)),
            full((1, _D)),
            full((1, 8)),
            full((3 * _D, enc_in)),
            full((1, enc_in)),
        ],
        out_specs=[
            tok_spec(enc_in),
            tok_spec(_D),
            pl.BlockSpec((_TOK, 1), lambda i: (i, 0)),
            pl.BlockSpec((1, _M), lambda i: (0, 0)),
        ],
        out_shape=[
            jax.ShapeDtypeStruct((_T, enc_in), jnp.float32),
            jax.ShapeDtypeStruct((_T, _D), jnp.float32),
            jax.ShapeDtypeStruct((_T, 1), jnp.float32),
            jax.ShapeDtypeStruct((1, _M), jnp.float32),
        ],
        scratch_shapes=[
            pltpu.VMEM((enc_in, enc_in), jnp.float32),
            pltpu.VMEM((enc_in, enc_in), jnp.float32),
            pltpu.VMEM((_M, enc_in), jnp.float32),
            pltpu.VMEM((1, enc_in), jnp.float32),
        ],
        compiler_params=pltpu.CompilerParams(
            dimension_semantics=("arbitrary",),
        ),
    )(x2, bd, W_in, b_in2, W_tr, b_tr2, mem, mem_t, invn, msq, wsig, bsig,
      W_dec, b_dec2)

    out = out2.reshape(_N, _L, enc_in)
    season = season2.reshape(_N, _L, _D)
    sigma = sigma2.reshape(_N, _L)
    return (out, mem, season, losses[0, 0], losses[0, 1], losses[0, 2], sigma)


# speedup vs baseline: 8.3845x; 1.1357x over previous
import functools

import jax
import jax.numpy as jnp
import numpy as np
from jax.experimental import pallas as pl
from jax.experimental.pallas import tpu as pltpu

_D = 512
_M = 128
_L = 100
_N = 256
_TOPK = 8
_KWIN = 25
_NB = 8
_TOK = _NB * _L
_T = _N * _L


def _avg_matrix():
    pad = (_KWIN - 1) // 2
    src = np.clip(np.arange(_L + 2 * pad) - pad, 0, _L - 1)
    m = np.zeros((_L, _L), np.float32)
    for i in range(_L):
        for j in src[i:i + _KWIN]:
            m[i, j] += 1.0 / _KWIN
    return m


_BD = np.kron(np.eye(_NB, dtype=np.float32), _avg_matrix())


def _fused_body(x_ref, bd_ref, w_in_ref, b_in_ref, w_tr_ref, b_tr_ref,
                mem_ref, mem_t_ref, invn_ref, msq_ref, wsig_ref, bsig_ref,
                w_dec_ref, b_dec_ref,
                out_ref, season_ref, sigma_ref, loss_ref,
                a1_ref, a2_ref, mw_ref, bias_ref):
    f32 = jnp.float32
    dot = functools.partial(jnp.dot, preferred_element_type=f32)
    xb = x_ref[...]
    w_in = w_in_ref[...]

    @pl.when(pl.program_id(0) == 0)
    def _():
        wd = w_dec_ref[...]
        wd1 = wd[0:_D]
        wd2 = wd[_D:2 * _D]
        wd3 = wd[2 * _D:3 * _D]
        w_tr = w_tr_ref[...]
        a1_ref[...] = dot(w_in, wd1)
        a2_ref[...] = dot(dot(w_in, w_tr), wd2)
        mw_ref[...] = dot(mem_ref[...], wd3)
        bias_ref[...] = (dot(b_in_ref[...], wd1)
                         + dot(dot(b_in_ref[...], w_tr) + b_tr_ref[...], wd2)
                         + b_dec_ref[...])
        loss_ref[...] = jnp.zeros_like(loss_ref)

    xavg = dot(bd_ref[...], xb)
    season = dot(xb - xavg, w_in)
    norm = jnp.sqrt(jnp.sum(season * season, axis=1, keepdims=True))
    q = season / (norm + 1e-12)
    season_ref[...] = q

    att_raw = dot(q, mem_t_ref[...])
    att = att_raw * 10.0

    row_i = jax.lax.broadcasted_iota(jnp.int32, (_M, _M), 0)
    col_i = jax.lax.broadcasted_iota(jnp.int32, (_M, _M), 1)
    ltri = jnp.where(row_i < col_i, 1.0, 0.0).astype(f32)

    cur = att
    topmask = jnp.zeros((_TOK, _M), dtype=jnp.bool_)
    amax = jnp.max(att, axis=1, keepdims=True)
    mval = amax
    for it in range(_TOPK):
        eqm = cur == mval
        eqf = jnp.where(eqm, 1.0, 0.0)
        pc = dot(eqf, ltri)
        sel = jnp.logical_and(eqm, pc < 0.5)
        topmask = jnp.logical_or(topmask, sel)
        cur = jnp.where(sel, -jnp.inf, cur)
        if it + 1 < _TOPK:
            mval = jnp.max(cur, axis=1, keepdims=True)

    p = jnp.where(topmask, jnp.exp(att - amax), 0.0)
    p = p / jnp.sum(p, axis=1, keepdims=True)

    out = dot(xb, a1_ref[...]) + dot(xavg, a2_ref[...]) + dot(p, mw_ref[...])
    out_ref[...] = out + bias_ref[...]

    sig = jnp.sum(q * wsig_ref[...], axis=1, keepdims=True) + bsig_ref[0, 0]
    sigma_ref[...] = sig

    sim = att_raw * invn_ref[...]
    smax = jnp.max(sim, axis=1, keepdims=True)
    lse = jnp.log(jnp.sum(jnp.exp(sim - smax), axis=1, keepdims=True)) + smax
    closs = jnp.sum(lse - smax) * (1.0 / _T)

    eqs = sim == smax
    eqsf = jnp.where(eqs, 1.0, 0.0)
    lsel = jnp.logical_and(eqs, dot(eqsf, ltri) < 0.5)
    att_sel = jnp.sum(jnp.where(lsel, att_raw, 0.0), axis=1, keepdims=True)
    msq_sel = jnp.sum(jnp.where(lsel, msq_ref[...], 0.0), axis=1, keepdims=True)
    qsq = jnp.sum(q * q, axis=1, keepdims=True)
    gloss = jnp.sum(qsq - 2.0 * att_sel + msq_sel) * (1.0 / (_T * _D))

    sig2 = sig * sig + 1e-6
    kld = jnp.sum(sig2 - jnp.log(sig2) - 1.0) * (0.5 / _T)

    li = jax.lax.broadcasted_iota(jnp.int32, (1, _M), 1)
    vals = (jnp.where(li == 0, closs, 0.0) + jnp.where(li == 1, gloss, 0.0)
            + jnp.where(li == 2, kld, 0.0))
    loss_ref[...] += vals


@functools.partial(jax.jit)
def kernel(x, W_in, b_in, W_tr, b_tr, mem, W_sig, b_sig, W_dec, b_dec):
    x2 = x.reshape(_T, -1)
    enc_in = x2.shape[1]
    bd = jnp.asarray(_BD)
    mem_t = mem.T
    mnorm = jnp.sqrt(jnp.sum(mem * mem, axis=1))
    invn = (10.0 / (mnorm + 1e-12)).reshape(1, _M)
    msq = (mnorm * mnorm).reshape(1, _M)
    wsig = W_sig.reshape(1, _D)
    bsig = jnp.broadcast_to(b_sig.reshape(1, 1), (1, 8))
    b_in2 = b_in.reshape(1, _D)
    b_tr2 = b_tr.reshape(1, _D)
    b_dec2 = b_dec.reshape(1, -1)

    grid = (_T // _TOK,)
    tok_spec = lambda cols: pl.BlockSpec((_TOK, cols), lambda i: (i, 0))
    full = lambda shape: pl.BlockSpec(shape, lambda i: (0,) * len(shape))

    out2, season2, sigma2, losses = pl.pallas_call(
        _fused_body,
        grid=grid,
        in_specs=[
            tok_spec(enc_in),
            full((_TOK, _TOK)),
            full((enc_in, _D)),
            full((1, _D)),
            full((_D, _D)),
            full((1, _D)),
            full((_M, _D)),
            full((_D, _M)),
            full((1, _M)),
            full((1, _M)),
            full((1, _D)),
            full((1, 8)),
            full((3 * _D, enc_in)),
            full((1, enc_in)),
        ],
        out_specs=[
            tok_spec(enc_in),
            tok_spec(_D),
            pl.BlockSpec((_TOK, 1), lambda i: (i, 0)),
            pl.BlockSpec((1, _M), lambda i: (0, 0)),
        ],
        out_shape=[
            jax.ShapeDtypeStruct((_T, enc_in), jnp.float32),
            jax.ShapeDtypeStruct((_T, _D), jnp.float32),
            jax.ShapeDtypeStruct((_T, 1), jnp.float32),
            jax.ShapeDtypeStruct((1, _M), jnp.float32),
        ],
        scratch_shapes=[
            pltpu.VMEM((enc_in, enc_in), jnp.float32),
            pltpu.VMEM((enc_in, enc_in), jnp.float32),
            pltpu.VMEM((_M, enc_in), jnp.float32),
            pltpu.VMEM((1, enc_in), jnp.float32),
        ],
        compiler_params=pltpu.CompilerParams(
            dimension_semantics=("arbitrary",),
        ),
    )(x2, bd, W_in, b_in2, W_tr, b_tr2, mem, mem_t, invn, msq, wsig, bsig,
      W_dec, b_dec2)

    out = out2.reshape(_N, _L, enc_in)
    season = season2.reshape(_N, _L, _D)
    sigma = sigma2.reshape(_N, _L)
    return (out, mem, season, losses[0, 0], losses[0, 1], losses[0, 2], sigma)


# speedup vs baseline: 9.5206x; 1.1355x over previous
import functools

import jax
import jax.numpy as jnp
import numpy as np
from jax.experimental import pallas as pl
from jax.experimental.pallas import tpu as pltpu

_D = 512
_M = 128
_L = 100
_N = 256
_TOPK = 8
_KWIN = 25
_NB = 8
_TOK = _NB * _L
_T = _N * _L


def _avg_matrix():
    pad = (_KWIN - 1) // 2
    src = np.clip(np.arange(_L + 2 * pad) - pad, 0, _L - 1)
    m = np.zeros((_L, _L), np.float32)
    for i in range(_L):
        for j in src[i:i + _KWIN]:
            m[i, j] += 1.0 / _KWIN
    return m


_BD = np.kron(np.eye(_NB, dtype=np.float32), _avg_matrix())


def _fused_body(x_ref, bd_ref, w_in_ref, b_in_ref, w_tr_ref, b_tr_ref,
                mem_ref, mem_t_ref, invn_ref, msq_ref, wsig_ref, bsig_ref,
                w_dec_ref, b_dec_ref,
                out_ref, season_ref, sigma_ref, loss_ref,
                a1_ref, a2_ref, mw_ref, bias_ref):
    f32 = jnp.float32
    dot = functools.partial(jnp.dot, preferred_element_type=f32)
    xb = x_ref[...].reshape(_TOK, -1)
    w_in = w_in_ref[...]

    @pl.when(pl.program_id(0) == 0)
    def _():
        wd = w_dec_ref[...]
        wd1 = wd[0:_D]
        wd2 = wd[_D:2 * _D]
        wd3 = wd[2 * _D:3 * _D]
        w_tr = w_tr_ref[...]
        a1_ref[...] = dot(w_in, wd1)
        a2_ref[...] = dot(dot(w_in, w_tr), wd2)
        mw_ref[...] = dot(mem_ref[...], wd3)
        bias_ref[...] = (dot(b_in_ref[...], wd1)
                         + dot(dot(b_in_ref[...], w_tr) + b_tr_ref[...], wd2)
                         + b_dec_ref[...])
        loss_ref[...] = jnp.zeros_like(loss_ref)

    xavg = dot(bd_ref[...], xb)
    season = dot(xb - xavg, w_in)
    norm = jnp.sqrt(jnp.sum(season * season, axis=1, keepdims=True))
    q = season / (norm + 1e-12)
    season_ref[...] = q.reshape(_NB, _L, _D)

    att_raw = dot(q, mem_t_ref[...])
    att = att_raw * 10.0

    row_i = jax.lax.broadcasted_iota(jnp.int32, (_M, _M), 0)
    col_i = jax.lax.broadcasted_iota(jnp.int32, (_M, _M), 1)
    ltri = jnp.where(row_i < col_i, 1.0, 0.0).astype(f32)

    cur = att
    topmask = jnp.zeros((_TOK, _M), dtype=jnp.bool_)
    amax = jnp.max(att, axis=1, keepdims=True)
    mval = amax
    for it in range(_TOPK):
        eqm = cur == mval
        eqf = jnp.where(eqm, 1.0, 0.0)
        pc = dot(eqf, ltri)
        sel = jnp.logical_and(eqm, pc < 0.5)
        topmask = jnp.logical_or(topmask, sel)
        cur = jnp.where(sel, -jnp.inf, cur)
        if it + 1 < _TOPK:
            mval = jnp.max(cur, axis=1, keepdims=True)

    p = jnp.where(topmask, jnp.exp(att - amax), 0.0)
    p = p / jnp.sum(p, axis=1, keepdims=True)

    out = dot(xb, a1_ref[...]) + dot(xavg, a2_ref[...]) + dot(p, mw_ref[...])
    out_ref[...] = (out + bias_ref[...]).reshape(_NB, _L, -1)

    sig = jnp.sum(q * wsig_ref[...], axis=1, keepdims=True) + bsig_ref[0, 0]
    sigma_ref[...] = sig.reshape(_NB, _L)

    sim = att_raw * invn_ref[...]
    smax = jnp.max(sim, axis=1, keepdims=True)
    lse = jnp.log(jnp.sum(jnp.exp(sim - smax), axis=1, keepdims=True)) + smax
    closs = jnp.sum(lse - smax) * (1.0 / _T)

    eqs = sim == smax
    eqsf = jnp.where(eqs, 1.0, 0.0)
    lsel = jnp.logical_and(eqs, dot(eqsf, ltri) < 0.5)
    att_sel = jnp.sum(jnp.where(lsel, att_raw, 0.0), axis=1, keepdims=True)
    msq_sel = jnp.sum(jnp.where(lsel, msq_ref[...], 0.0), axis=1, keepdims=True)
    qsq = jnp.sum(q * q, axis=1, keepdims=True)
    gloss = jnp.sum(qsq - 2.0 * att_sel + msq_sel) * (1.0 / (_T * _D))

    sig2 = sig * sig + 1e-6
    kld = jnp.sum(sig2 - jnp.log(sig2) - 1.0) * (0.5 / _T)

    li = jax.lax.broadcasted_iota(jnp.int32, (1, _M), 1)
    vals = (jnp.where(li == 0, closs, 0.0) + jnp.where(li == 1, gloss, 0.0)
            + jnp.where(li == 2, kld, 0.0))
    loss_ref[...] += vals


@functools.partial(jax.jit)
def kernel(x, W_in, b_in, W_tr, b_tr, mem, W_sig, b_sig, W_dec, b_dec):
    enc_in = x.shape[2]
    bd = jnp.asarray(_BD)
    mem_t = mem.T
    mnorm = jnp.sqrt(jnp.sum(mem * mem, axis=1))
    invn = (10.0 / (mnorm + 1e-12)).reshape(1, _M)
    msq = (mnorm * mnorm).reshape(1, _M)
    wsig = W_sig.reshape(1, _D)
    bsig = jnp.broadcast_to(b_sig.reshape(1, 1), (1, 8))
    b_in2 = b_in.reshape(1, _D)
    b_tr2 = b_tr.reshape(1, _D)
    b_dec2 = b_dec.reshape(1, -1)

    grid = (_T // _TOK,)
    tok3 = lambda cols: pl.BlockSpec((_NB, _L, cols), lambda i: (i, 0, 0))
    full = lambda shape: pl.BlockSpec(shape, lambda i: (0,) * len(shape))

    out3, season3, sigma2, losses = pl.pallas_call(
        _fused_body,
        grid=grid,
        in_specs=[
            tok3(enc_in),
            full((_TOK, _TOK)),
            full((enc_in, _D)),
            full((1, _D)),
            full((_D, _D)),
            full((1, _D)),
            full((_M, _D)),
            full((_D, _M)),
            full((1, _M)),
            full((1, _M)),
            full((1, _D)),
            full((1, 8)),
            full((3 * _D, enc_in)),
            full((1, enc_in)),
        ],
        out_specs=[
            tok3(enc_in),
            tok3(_D),
            pl.BlockSpec((_NB, _L), lambda i: (i, 0)),
            pl.BlockSpec((1, _M), lambda i: (0, 0)),
        ],
        out_shape=[
            jax.ShapeDtypeStruct((_N, _L, enc_in), jnp.float32),
            jax.ShapeDtypeStruct((_N, _L, _D), jnp.float32),
            jax.ShapeDtypeStruct((_N, _L), jnp.float32),
            jax.ShapeDtypeStruct((1, _M), jnp.float32),
        ],
        scratch_shapes=[
            pltpu.VMEM((enc_in, enc_in), jnp.float32),
            pltpu.VMEM((enc_in, enc_in), jnp.float32),
            pltpu.VMEM((_M, enc_in), jnp.float32),
            pltpu.VMEM((1, enc_in), jnp.float32),
        ],
        compiler_params=pltpu.CompilerParams(
            dimension_semantics=("arbitrary",),
        ),
    )(x, bd, W_in, b_in2, W_tr, b_tr2, mem, mem_t, invn, msq, wsig, bsig,
      W_dec, b_dec2)

    return (out3, mem, season3, losses[0, 0], losses[0, 1], losses[0, 2],
            sigma2)


# speedup vs baseline: 10.5560x; 1.1087x over previous
import functools

import jax
import jax.numpy as jnp
import numpy as np
from jax.experimental import pallas as pl
from jax.experimental.pallas import tpu as pltpu

_D = 512
_M = 128
_L = 100
_N = 256
_TOPK = 8
_KWIN = 25
_NB = 8
_TOK = _NB * _L
_T = _N * _L


def _avg_matrix():
    pad = (_KWIN - 1) // 2
    src = np.clip(np.arange(_L + 2 * pad) - pad, 0, _L - 1)
    m = np.zeros((_L, _L), np.float32)
    for i in range(_L):
        for j in src[i:i + _KWIN]:
            m[i, j] += 1.0 / _KWIN
    return m


def _fused_body(x_ref, av_ref, w_in_ref, b_in_ref, w_tr_ref, b_tr_ref,
                mem_ref, mem_t_ref, invn_ref, msq_ref, wsig_ref, bsig_ref,
                w_dec_ref, b_dec_ref,
                out_ref, season_ref, sigma_ref, loss_ref,
                a1_ref, a2_ref, mw_ref, bias_ref):
    f32 = jnp.float32
    dot = functools.partial(jnp.dot, preferred_element_type=f32)
    xb3 = x_ref[...]
    xb = xb3.reshape(_TOK, -1)
    w_in = w_in_ref[...]

    @pl.when(pl.program_id(0) == 0)
    def _():
        wd = w_dec_ref[...]
        wd1 = wd[0:_D]
        wd2 = wd[_D:2 * _D]
        wd3 = wd[2 * _D:3 * _D]
        w_tr = w_tr_ref[...]
        a1_ref[...] = dot(w_in, wd1)
        a2_ref[...] = dot(dot(w_in, w_tr), wd2)
        mw_ref[...] = dot(mem_ref[...], wd3)
        bias_ref[...] = (dot(b_in_ref[...], wd1)
                         + dot(dot(b_in_ref[...], w_tr) + b_tr_ref[...], wd2)
                         + b_dec_ref[...])
        loss_ref[...] = jnp.zeros_like(loss_ref)

    av = av_ref[...]
    xavg3 = jnp.stack([dot(av, xb3[n]) for n in range(_NB)], axis=0)
    xavg = xavg3.reshape(_TOK, -1)
    season = dot(xb - xavg, w_in)

    ones_d = jnp.full((_D, 1), 1.0, f32)
    nsq = dot(season * season, ones_d)
    norm = jnp.sqrt(nsq)
    rnorm = 1.0 / (norm + 1e-12)
    q = season * rnorm
    season_ref[...] = q.reshape(_NB, _L, _D)

    att_raw = dot(q, mem_t_ref[...])
    art = jnp.transpose(att_raw)
    att = art * 10.0

    row_i = jax.lax.broadcasted_iota(jnp.int32, (_M, _M), 0)
    col_i = jax.lax.broadcasted_iota(jnp.int32, (_M, _M), 1)
    ltri = jnp.where(col_i < row_i, 1.0, 0.0).astype(f32)

    cur = att
    topmask = jnp.zeros((_M, _TOK), dtype=jnp.bool_)
    amax = jnp.max(att, axis=0, keepdims=True)
    mval = amax
    for it in range(_TOPK):
        eqm = cur == mval
        eqf = jnp.where(eqm, 1.0, 0.0)
        pc = dot(ltri, eqf)
        sel = jnp.logical_and(eqm, pc < 0.5)
        topmask = jnp.logical_or(topmask, sel)
        cur = jnp.where(sel, -jnp.inf, cur)
        if it + 1 < _TOPK:
            mval = jnp.max(cur, axis=0, keepdims=True)

    pt = jnp.where(topmask, jnp.exp(att - amax), 0.0)
    pt = pt / jnp.sum(pt, axis=0, keepdims=True)
    p = jnp.transpose(pt)

    out = dot(xb, a1_ref[...]) + dot(xavg, a2_ref[...]) + dot(p, mw_ref[...])
    out_ref[...] = (out + bias_ref[...]).reshape(_NB, _L, -1)

    sig = dot(q, wsig_ref[...]) + bsig_ref[0, 0]
    sigma_ref[...] = sig.reshape(_NB, _L)

    sim = art * invn_ref[...]
    smax = jnp.max(sim, axis=0, keepdims=True)
    lse = jnp.log(jnp.sum(jnp.exp(sim - smax), axis=0, keepdims=True)) + smax
    closs = jnp.sum(lse - smax) * (1.0 / _T)

    eqs = sim == smax
    eqsf = jnp.where(eqs, 1.0, 0.0)
    lsel = jnp.logical_and(eqs, dot(ltri, eqsf) < 0.5)
    att_sel = jnp.sum(jnp.where(lsel, art, 0.0))
    msq_sel = jnp.sum(jnp.where(lsel, msq_ref[...], 0.0))
    qsq = jnp.sum(nsq * rnorm * rnorm)
    gloss = (qsq - 2.0 * att_sel + msq_sel) * (1.0 / (_T * _D))

    sig2 = sig * sig + 1e-6
    kld = jnp.sum(sig2 - jnp.log(sig2) - 1.0) * (0.5 / _T)

    li = jax.lax.broadcasted_iota(jnp.int32, (1, _M), 1)
    vals = (jnp.where(li == 0, closs, 0.0) + jnp.where(li == 1, gloss, 0.0)
            + jnp.where(li == 2, kld, 0.0))
    loss_ref[...] += vals


@functools.partial(jax.jit)
def kernel(x, W_in, b_in, W_tr, b_tr, mem, W_sig, b_sig, W_dec, b_dec):
    enc_in = x.shape[2]
    av = jnp.asarray(_avg_matrix())
    mem_t = mem.T
    mnorm = jnp.sqrt(jnp.sum(mem * mem, axis=1))
    invn = (10.0 / (mnorm + 1e-12)).reshape(_M, 1)
    msq = (mnorm * mnorm).reshape(_M, 1)
    bsig = b_sig.reshape(1, 1)
    b_in2 = b_in.reshape(1, _D)
    b_tr2 = b_tr.reshape(1, _D)
    b_dec2 = b_dec.reshape(1, -1)

    grid = (_T // _TOK,)
    tok3 = lambda cols: pl.BlockSpec((_NB, _L, cols), lambda i: (i, 0, 0))
    full = lambda shape: pl.BlockSpec(shape, lambda i: (0,) * len(shape))

    out3, season3, sigma2, losses = pl.pallas_call(
        _fused_body,
        grid=grid,
        in_specs=[
            tok3(enc_in),
            full((_L, _L)),
            full((enc_in, _D)),
            full((1, _D)),
            full((_D, _D)),
            full((1, _D)),
            full((_M, _D)),
            full((_D, _M)),
            full((_M, 1)),
            full((_M, 1)),
            full((_D, 1)),
            full((1, 1)),
            full((3 * _D, enc_in)),
            full((1, enc_in)),
        ],
        out_specs=[
            tok3(enc_in),
            tok3(_D),
            pl.BlockSpec((_NB, _L), lambda i: (i, 0)),
            pl.BlockSpec((1, _M), lambda i: (0, 0)),
        ],
        out_shape=[
            jax.ShapeDtypeStruct((_N, _L, enc_in), jnp.float32),
            jax.ShapeDtypeStruct((_N, _L, _D), jnp.float32),
            jax.ShapeDtypeStruct((_N, _L), jnp.float32),
            jax.ShapeDtypeStruct((1, _M), jnp.float32),
        ],
        scratch_shapes=[
            pltpu.VMEM((enc_in, enc_in), jnp.float32),
            pltpu.VMEM((enc_in, enc_in), jnp.float32),
            pltpu.VMEM((_M, enc_in), jnp.float32),
            pltpu.VMEM((1, enc_in), jnp.float32),
        ],
        compiler_params=pltpu.CompilerParams(
            dimension_semantics=("arbitrary",),
        ),
    )(x, av, W_in, b_in2, W_tr, b_tr2, mem, mem_t, invn, msq, W_sig, bsig,
      W_dec, b_dec2)

    return (out3, mem, season3, losses[0, 0], losses[0, 1], losses[0, 2],
            sigma2)


# speedup vs baseline: 10.7986x; 1.0230x over previous
import functools

import jax
import jax.numpy as jnp
import numpy as np
from jax.experimental import pallas as pl
from jax.experimental.pallas import tpu as pltpu

_D = 512
_M = 128
_L = 100
_N = 256
_TOPK = 8
_KWIN = 25
_NB = 8
_TOK = _NB * _L
_T = _N * _L


def _avg_matrix():
    pad = (_KWIN - 1) // 2
    src = np.clip(np.arange(_L + 2 * pad) - pad, 0, _L - 1)
    m = np.zeros((_L, _L), np.float32)
    for i in range(_L):
        for j in src[i:i + _KWIN]:
            m[i, j] += 1.0 / _KWIN
    return m


def _fused_body(x_ref, av_ref, w_in_ref, b_in_ref, w_tr_ref, b_tr_ref,
                mem_ref, wsig_ref, bsig_ref,
                w_dec_ref, b_dec_ref,
                out_ref, season_ref, sigma_ref, loss_ref,
                a1_ref, a2_ref, mw_ref, bias_ref, mem_t_ref, invn_ref,
                msq_ref):
    f32 = jnp.float32
    dot = functools.partial(jnp.dot, preferred_element_type=f32)
    xb3 = x_ref[...]
    xb = xb3.reshape(_TOK, -1)
    w_in = w_in_ref[...]
    ones_d = jnp.full((_D, 1), 1.0, f32)

    @pl.when(pl.program_id(0) == 0)
    def _():
        wd = w_dec_ref[...]
        wd1 = wd[0:_D]
        wd2 = wd[_D:2 * _D]
        wd3 = wd[2 * _D:3 * _D]
        w_tr = w_tr_ref[...]
        mem = mem_ref[...]
        a1_ref[...] = dot(w_in, wd1)
        a2_ref[...] = dot(dot(w_in, w_tr), wd2)
        mw_ref[...] = dot(mem, wd3)
        bias_ref[...] = (dot(b_in_ref[...], wd1)
                         + dot(dot(b_in_ref[...], w_tr) + b_tr_ref[...], wd2)
                         + b_dec_ref[...])
        mem_t_ref[...] = jnp.transpose(mem)
        mnsq = dot(mem * mem, ones_d)
        invn_ref[...] = 10.0 / (jnp.sqrt(mnsq) + 1e-12)
        msq_ref[...] = mnsq
        loss_ref[...] = jnp.zeros_like(loss_ref)

    av = av_ref[...]
    xavg3 = jnp.stack([dot(av, xb3[n]) for n in range(_NB)], axis=0)
    xavg = xavg3.reshape(_TOK, -1)
    season = dot(xb - xavg, w_in)

    nsq = dot(season * season, ones_d)
    norm = jnp.sqrt(nsq)
    rnorm = 1.0 / (norm + 1e-12)
    q = season * rnorm
    season_ref[...] = q.reshape(_NB, _L, _D)

    att_raw = dot(q, mem_t_ref[...])
    art = jnp.transpose(att_raw)
    att = art * 10.0

    row_i = jax.lax.broadcasted_iota(jnp.int32, (_M, _M), 0)
    col_i = jax.lax.broadcasted_iota(jnp.int32, (_M, _M), 1)
    ltri = jnp.where(col_i < row_i, 1.0, 0.0).astype(f32)

    cur = att
    topmask = jnp.zeros((_M, _TOK), dtype=jnp.bool_)
    amax = jnp.max(att, axis=0, keepdims=True)
    mval = amax
    for it in range(_TOPK):
        eqm = cur == mval
        eqf = jnp.where(eqm, 1.0, 0.0)
        pc = dot(ltri, eqf)
        sel = jnp.logical_and(eqm, pc < 0.5)
        topmask = jnp.logical_or(topmask, sel)
        cur = jnp.where(sel, -jnp.inf, cur)
        if it + 1 < _TOPK:
            mval = jnp.max(cur, axis=0, keepdims=True)

    pt = jnp.where(topmask, jnp.exp(att - amax), 0.0)
    pt = pt / jnp.sum(pt, axis=0, keepdims=True)
    p = jnp.transpose(pt)

    out = dot(xb, a1_ref[...]) + dot(xavg, a2_ref[...]) + dot(p, mw_ref[...])
    out_ref[...] = (out + bias_ref[...]).reshape(_NB, _L, -1)

    sig = dot(q, wsig_ref[...]) + bsig_ref[0, 0]
    sigma_ref[...] = sig.reshape(_NB, _L)

    sim = art * invn_ref[...]
    smax = jnp.max(sim, axis=0, keepdims=True)
    lse = jnp.log(jnp.sum(jnp.exp(sim - smax), axis=0, keepdims=True)) + smax
    closs = jnp.sum(lse - smax) * (1.0 / _T)

    eqs = sim == smax
    eqsf = jnp.where(eqs, 1.0, 0.0)
    lsel = jnp.logical_and(eqs, dot(ltri, eqsf) < 0.5)
    att_sel = jnp.sum(jnp.where(lsel, art, 0.0))
    msq_sel = jnp.sum(jnp.where(lsel, msq_ref[...], 0.0))
    qsq = jnp.sum(nsq * rnorm * rnorm)
    gloss = (qsq - 2.0 * att_sel + msq_sel) * (1.0 / (_T * _D))

    sig2 = sig * sig + 1e-6
    kld = jnp.sum(sig2 - jnp.log(sig2) - 1.0) * (0.5 / _T)

    li = jax.lax.broadcasted_iota(jnp.int32, (1, _M), 1)
    vals = (jnp.where(li == 0, closs, 0.0) + jnp.where(li == 1, gloss, 0.0)
            + jnp.where(li == 2, kld, 0.0))
    loss_ref[...] += vals


@functools.partial(jax.jit)
def kernel(x, W_in, b_in, W_tr, b_tr, mem, W_sig, b_sig, W_dec, b_dec):
    enc_in = x.shape[2]
    av = jnp.asarray(_avg_matrix())
    bsig = b_sig.reshape(1, 1)
    b_in2 = b_in.reshape(1, _D)
    b_tr2 = b_tr.reshape(1, _D)
    b_dec2 = b_dec.reshape(1, -1)

    grid = (_T // _TOK,)
    tok3 = lambda cols: pl.BlockSpec((_NB, _L, cols), lambda i: (i, 0, 0))
    full = lambda shape: pl.BlockSpec(shape, lambda i: (0,) * len(shape))

    out3, season3, sigma2, losses = pl.pallas_call(
        _fused_body,
        grid=grid,
        in_specs=[
            tok3(enc_in),
            full((_L, _L)),
            full((enc_in, _D)),
            full((1, _D)),
            full((_D, _D)),
            full((1, _D)),
            full((_M, _D)),
            full((_D, 1)),
            full((1, 1)),
            full((3 * _D, enc_in)),
            full((1, enc_in)),
        ],
        out_specs=[
            tok3(enc_in),
            tok3(_D),
            pl.BlockSpec((_NB, _L), lambda i: (i, 0)),
            pl.BlockSpec((1, _M), lambda i: (0, 0)),
        ],
        out_shape=[
            jax.ShapeDtypeStruct((_N, _L, enc_in), jnp.float32),
            jax.ShapeDtypeStruct((_N, _L, _D), jnp.float32),
            jax.ShapeDtypeStruct((_N, _L), jnp.float32),
            jax.ShapeDtypeStruct((1, _M), jnp.float32),
        ],
        scratch_shapes=[
            pltpu.VMEM((enc_in, enc_in), jnp.float32),
            pltpu.VMEM((enc_in, enc_in), jnp.float32),
            pltpu.VMEM((_M, enc_in), jnp.float32),
            pltpu.VMEM((1, enc_in), jnp.float32),
            pltpu.VMEM((_D, _M), jnp.float32),
            pltpu.VMEM((_M, 1), jnp.float32),
            pltpu.VMEM((_M, 1), jnp.float32),
        ],
        compiler_params=pltpu.CompilerParams(
            dimension_semantics=("arbitrary",),
        ),
    )(x, av, W_in, b_in2, W_tr, b_tr2, mem, W_sig, bsig, W_dec, b_dec2)

    return (out3, mem, season3, losses[0, 0], losses[0, 1], losses[0, 2],
            sigma2)


# speedup vs baseline: 10.8309x; 1.0030x over previous
import functools

import jax
import jax.numpy as jnp
import numpy as np
from jax.experimental import pallas as pl
from jax.experimental.pallas import tpu as pltpu

_D = 512
_M = 128
_L = 100
_N = 256
_TOPK = 8
_KWIN = 25
_NB = 8
_TOK = _NB * _L
_T = _N * _L


def _avg_matrix():
    pad = (_KWIN - 1) // 2
    src = np.clip(np.arange(_L + 2 * pad) - pad, 0, _L - 1)
    m = np.zeros((_L, _L), np.float32)
    for i in range(_L):
        for j in src[i:i + _KWIN]:
            m[i, j] += 1.0 / _KWIN
    return m


def _perms():
    p = np.zeros((_TOK, _TOK), np.float32)
    for n in range(_NB):
        for l in range(_L):
            p[l * _NB + n, n * _L + l] = 1.0
    bd = np.kron(np.eye(_NB, dtype=np.float32), _avg_matrix())
    return p, p @ bd, p.T


_P, _PBD, _PT = _perms()


def _fused_body(x_ref, av_ref, pm_ref, pbd_ref, ptm_ref,
                w_in_ref, b_in_ref, w_tr_ref, b_tr_ref,
                mem_ref, wsig_ref, bsig_ref, w_dect_ref, b_dec_ref,
                out_ref, season_ref, sigma_ref, loss_ref,
                a1_ref, a2_ref, mw_ref, bias_ref, mem_t_ref, invn_ref,
                msq_ref):
    f32 = jnp.float32
    dot = functools.partial(jnp.dot, preferred_element_type=f32)

    def dot_t(a, b):
        return jax.lax.dot_general(a, b, (((1,), (1,)), ((), ())),
                                   preferred_element_type=f32)

    xb3 = x_ref[...]
    xb = xb3.reshape(_TOK, -1)
    w_in = w_in_ref[...]
    ones_d = jnp.full((_D, 1), 1.0, f32)

    @pl.when(pl.program_id(0) == 0)
    def _():
        wdt = w_dect_ref[...]
        w_tr = w_tr_ref[...]
        mem = mem_ref[...]
        a1_ref[...] = dot_t(w_in, wdt[:, 0:_D])
        a2_ref[...] = dot_t(dot(w_in, w_tr), wdt[:, _D:2 * _D])
        mw_ref[...] = dot_t(mem, wdt[:, 2 * _D:3 * _D])
        bias_ref[...] = (dot_t(b_in_ref[...], wdt[:, 0:_D])
                         + dot_t(dot(b_in_ref[...], w_tr) + b_tr_ref[...],
                                 wdt[:, _D:2 * _D])
                         + b_dec_ref[...])
        mem_t_ref[...] = jnp.transpose(mem)
        mnsq = dot(mem * mem, ones_d)
        invn_ref[...] = 10.0 / (jnp.sqrt(mnsq) + 1e-12)
        msq_ref[...] = mnsq
        loss_ref[...] = jnp.zeros_like(loss_ref)

    av = av_ref[...]
    xavg3 = jnp.stack([dot(av, xb3[n]) for n in range(_NB)], axis=0)
    xavg = xavg3.reshape(_TOK, -1)

    xb_lm = dot(pm_ref[...], xb)
    xavg_lm = dot(pbd_ref[...], xb)
    season = dot(xb_lm - xavg_lm, w_in)

    nsq = dot(season * season, ones_d)
    norm = jnp.sqrt(nsq)
    rnorm = 1.0 / (norm + 1e-12)
    q = season * rnorm
    season_ref[...] = q.reshape(_L, _NB, _D)

    att_raw = dot(q, mem_t_ref[...])
    art = jnp.transpose(att_raw)
    att = art * 10.0

    row_i = jax.lax.broadcasted_iota(jnp.int32, (_M, _M), 0)
    col_i = jax.lax.broadcasted_iota(jnp.int32, (_M, _M), 1)
    ltri = jnp.where(col_i < row_i, 1.0, 0.0).astype(f32)

    cur = att
    topmask = jnp.zeros((_M, _TOK), dtype=jnp.bool_)
    amax = jnp.max(att, axis=0, keepdims=True)
    mval = amax
    for it in range(_TOPK):
        eqm = cur == mval
        eqf = jnp.where(eqm, 1.0, 0.0)
        pc = dot(ltri, eqf)
        sel = jnp.logical_and(eqm, pc < 0.5)
        topmask = jnp.logical_or(topmask, sel)
        cur = jnp.where(sel, -jnp.inf, cur)
        if it + 1 < _TOPK:
            mval = jnp.max(cur, axis=0, keepdims=True)

    pt = jnp.where(topmask, jnp.exp(att - amax), 0.0)
    pt = pt / jnp.sum(pt, axis=0, keepdims=True)
    p = jnp.transpose(pt)

    read_nm = dot(ptm_ref[...], dot(p, mw_ref[...]))
    out = dot(xb, a1_ref[...]) + dot(xavg, a2_ref[...]) + read_nm
    out_ref[...] = (out + bias_ref[...]).reshape(_NB, _L, -1)

    sig = dot(q * wsig_ref[...], ones_d) + bsig_ref[0, 0]
    sigma_ref[...] = sig.reshape(_L, _NB, 1)

    sim = art * invn_ref[...]
    smax = jnp.max(sim, axis=0, keepdims=True)
    lse = jnp.log(jnp.sum(jnp.exp(sim - smax), axis=0, keepdims=True)) + smax
    closs = jnp.sum(lse - smax) * (1.0 / _T)

    eqs = sim == smax
    eqsf = jnp.where(eqs, 1.0, 0.0)
    lsel = jnp.logical_and(eqs, dot(ltri, eqsf) < 0.5)
    att_sel = jnp.sum(jnp.where(lsel, art, 0.0))
    msq_sel = jnp.sum(jnp.where(lsel, msq_ref[...], 0.0))
    qsq = jnp.sum(nsq * rnorm * rnorm)
    gloss = (qsq - 2.0 * att_sel + msq_sel) * (1.0 / (_T * _D))

    sig2 = sig * sig + 1e-6
    kld = jnp.sum(sig2 - jnp.log(sig2) - 1.0) * (0.5 / _T)

    li = jax.lax.broadcasted_iota(jnp.int32, (1, _M), 1)
    vals = (jnp.where(li == 0, closs, 0.0) + jnp.where(li == 1, gloss, 0.0)
            + jnp.where(li == 2, kld, 0.0))
    loss_ref[...] += vals


@functools.partial(jax.jit)
def kernel(x, W_in, b_in, W_tr, b_tr, mem, W_sig, b_sig, W_dec, b_dec):
    enc_in = x.shape[2]
    av = jnp.asarray(_avg_matrix())
    pm = jnp.asarray(_P)
    pbd = jnp.asarray(_PBD)
    ptm = jnp.asarray(_PT)
    wsig_row = W_sig.T
    wdec_t = W_dec.T
    bsig = b_sig.reshape(1, 1)
    b_in2 = b_in.reshape(1, _D)
    b_tr2 = b_tr.reshape(1, _D)
    b_dec2 = b_dec.reshape(1, -1)

    grid = (_T // _TOK,)
    tok3 = lambda cols: pl.BlockSpec((_NB, _L, cols), lambda i: (i, 0, 0))
    full = lambda shape: pl.BlockSpec(shape, lambda i: (0,) * len(shape))

    out3, seasonL, sigmaL, losses = pl.pallas_call(
        _fused_body,
        grid=grid,
        in_specs=[
            tok3(enc_in),
            full((_L, _L)),
            full((_TOK, _TOK)),
            full((_TOK, _TOK)),
            full((_TOK, _TOK)),
            full((enc_in, _D)),
            full((1, _D)),
            full((_D, _D)),
            full((1, _D)),
            full((_M, _D)),
            full((1, _D)),
            full((1, 1)),
            full((enc_in, 3 * _D)),
            full((1, enc_in)),
        ],
        out_specs=[
            tok3(enc_in),
            pl.BlockSpec((_L, _NB, _D), lambda i: (0, i, 0)),
            pl.BlockSpec((_L, _NB, 1), lambda i: (0, i, 0)),
            pl.BlockSpec((1, _M), lambda i: (0, 0)),
        ],
        out_shape=[
            jax.ShapeDtypeStruct((_N, _L, enc_in), jnp.float32),
            jax.ShapeDtypeStruct((_L, _N, _D), jnp.float32),
            jax.ShapeDtypeStruct((_L, _N, 1), jnp.float32),
            jax.ShapeDtypeStruct((1, _M), jnp.float32),
        ],
        scratch_shapes=[
            pltpu.VMEM((enc_in, enc_in), jnp.float32),
            pltpu.VMEM((enc_in, enc_in), jnp.float32),
            pltpu.VMEM((_M, enc_in), jnp.float32),
            pltpu.VMEM((1, enc_in), jnp.float32),
            pltpu.VMEM((_D, _M), jnp.float32),
            pltpu.VMEM((_M, 1), jnp.float32),
            pltpu.VMEM((_M, 1), jnp.float32),
        ],
        compiler_params=pltpu.CompilerParams(
            dimension_semantics=("arbitrary",),
        ),
    )(x, av, pm, pbd, ptm, W_in, b_in2, W_tr, b_tr2, mem, wsig_row, bsig,
      wdec_t, b_dec2)

    season = seasonL.transpose(1, 0, 2)
    sigma = sigmaL.reshape(_L, _N).T
    return (out3, mem, season, losses[0, 0], losses[0, 1], losses[0, 2],
            sigma)


# speedup vs baseline: 13.0775x; 1.2074x over previous
import functools

import jax
import jax.numpy as jnp
import numpy as np
from jax.experimental import pallas as pl
from jax.experimental.pallas import tpu as pltpu

_D = 512
_M = 128
_L = 100
_N = 256
_TOPK = 8
_KWIN = 25
_NB = 8
_TOK = _NB * _L
_T = _N * _L


def _avg_matrix():
    pad = (_KWIN - 1) // 2
    src = np.clip(np.arange(_L + 2 * pad) - pad, 0, _L - 1)
    m = np.zeros((_L, _L), np.float32)
    for i in range(_L):
        for j in src[i:i + _KWIN]:
            m[i, j] += 1.0 / _KWIN
    return m


def _fused_body(x_ref, av_ref, w_in_ref, b_in_ref, w_tr_ref, b_tr_ref,
                mem_ref, wsig_ref, bsig_ref, w_dect_ref, b_dec_ref,
                out_ref, season_ref, sigma_ref, loss_ref,
                a1_ref, a2_ref, mw_ref, bias_ref, mem_t_ref, invn_ref,
                msq_ref):
    f32 = jnp.float32
    dot = functools.partial(jnp.dot, preferred_element_type=f32)

    def dot_t(a, b):
        return jax.lax.dot_general(a, b, (((1,), (1,)), ((), ())),
                                   preferred_element_type=f32)

    xb3 = x_ref[...]
    xb = xb3.reshape(_TOK, -1)
    w_in = w_in_ref[...]
    ones_d = jnp.full((_D, 1), 1.0, f32)

    @pl.when(pl.program_id(0) == 0)
    def _():
        wdt = w_dect_ref[...]
        w_tr = w_tr_ref[...]
        mem = mem_ref[...]
        a1_ref[...] = dot_t(w_in, wdt[:, 0:_D])
        a2_ref[...] = dot_t(dot(w_in, w_tr), wdt[:, _D:2 * _D])
        mw_ref[...] = dot_t(mem, wdt[:, 2 * _D:3 * _D])
        bias_ref[...] = (dot_t(b_in_ref[...], wdt[:, 0:_D])
                         + dot_t(dot(b_in_ref[...], w_tr) + b_tr_ref[...],
                                 wdt[:, _D:2 * _D])
                         + b_dec_ref[...])
        mem_t_ref[...] = jnp.transpose(mem)
        mnsq = dot(mem * mem, ones_d)
        invn_ref[...] = 10.0 / (jnp.sqrt(mnsq) + 1e-12)
        msq_ref[...] = mnsq
        loss_ref[...] = jnp.zeros_like(loss_ref)

    av = av_ref[...]
    xavg3 = jnp.stack([dot(av, xb3[n]) for n in range(_NB)], axis=0)
    xavg = xavg3.reshape(_TOK, -1)
    season = dot(xb - xavg, w_in)

    nsq = dot(season * season, ones_d)
    norm = jnp.sqrt(nsq)
    rnorm = 1.0 / (norm + 1e-12)
    q = season * rnorm
    for n in range(_NB):
        season_ref[:, n, :] = q[n * _L:(n + 1) * _L, :]

    att_raw = dot(q, mem_t_ref[...])
    art = jnp.transpose(att_raw)
    att = art * 10.0

    row_i = jax.lax.broadcasted_iota(jnp.int32, (_M, _M), 0)
    col_i = jax.lax.broadcasted_iota(jnp.int32, (_M, _M), 1)
    ltri = jnp.where(col_i < row_i, 1.0, 0.0).astype(f32)

    cur = att
    topmask = jnp.zeros((_M, _TOK), dtype=jnp.bool_)
    amax = jnp.max(att, axis=0, keepdims=True)
    mval = amax
    for it in range(_TOPK):
        eqm = cur == mval
        eqf = jnp.where(eqm, 1.0, 0.0)
        pc = dot(ltri, eqf)
        sel = jnp.logical_and(eqm, pc < 0.5)
        topmask = jnp.logical_or(topmask, sel)
        cur = jnp.where(sel, -jnp.inf, cur)
        if it + 1 < _TOPK:
            mval = jnp.max(cur, axis=0, keepdims=True)

    pt = jnp.where(topmask, jnp.exp(att - amax), 0.0)
    pt = pt / jnp.sum(pt, axis=0, keepdims=True)
    p = jnp.transpose(pt)

    out = dot(xb, a1_ref[...]) + dot(xavg, a2_ref[...]) + dot(p, mw_ref[...])
    out_ref[...] = (out + bias_ref[...]).reshape(_NB, _L, -1)

    sig = dot(q * wsig_ref[...], ones_d) + bsig_ref[0, 0]
    sigma_ref[...] = sig.reshape(_NB, _L)

    sim = art * invn_ref[...]
    smax = jnp.max(sim, axis=0, keepdims=True)
    lse = jnp.log(jnp.sum(jnp.exp(sim - smax), axis=0, keepdims=True)) + smax
    closs = jnp.sum(lse - smax) * (1.0 / _T)

    eqs = sim == smax
    eqsf = jnp.where(eqs, 1.0, 0.0)
    lsel = jnp.logical_and(eqs, dot(ltri, eqsf) < 0.5)
    att_sel = jnp.sum(jnp.where(lsel, art, 0.0))
    msq_sel = jnp.sum(jnp.where(lsel, msq_ref[...], 0.0))
    qsq = jnp.sum(nsq * rnorm * rnorm)
    gloss = (qsq - 2.0 * att_sel + msq_sel) * (1.0 / (_T * _D))

    sig2 = sig * sig + 1e-6
    kld = jnp.sum(sig2 - jnp.log(sig2) - 1.0) * (0.5 / _T)

    li = jax.lax.broadcasted_iota(jnp.int32, (1, _M), 1)
    vals = (jnp.where(li == 0, closs, 0.0) + jnp.where(li == 1, gloss, 0.0)
            + jnp.where(li == 2, kld, 0.0))
    loss_ref[...] += vals


@functools.partial(jax.jit)
def kernel(x, W_in, b_in, W_tr, b_tr, mem, W_sig, b_sig, W_dec, b_dec):
    enc_in = x.shape[2]
    av = jnp.asarray(_avg_matrix())
    wsig_row = W_sig.T
    wdec_t = W_dec.T
    bsig = b_sig.reshape(1, 1)
    b_in2 = b_in.reshape(1, _D)
    b_tr2 = b_tr.reshape(1, _D)
    b_dec2 = b_dec.reshape(1, -1)

    grid = (_T // _TOK,)
    tok3 = lambda cols: pl.BlockSpec((_NB, _L, cols), lambda i: (i, 0, 0))
    full = lambda shape: pl.BlockSpec(shape, lambda i: (0,) * len(shape))

    out3, seasonL, sigma2, losses = pl.pallas_call(
        _fused_body,
        grid=grid,
        in_specs=[
            tok3(enc_in),
            full((_L, _L)),
            full((enc_in, _D)),
            full((1, _D)),
            full((_D, _D)),
            full((1, _D)),
            full((_M, _D)),
            full((1, _D)),
            full((1, 1)),
            full((enc_in, 3 * _D)),
            full((1, enc_in)),
        ],
        out_specs=[
            tok3(enc_in),
            pl.BlockSpec((_L, _NB, _D), lambda i: (0, i, 0)),
            pl.BlockSpec((_NB, _L), lambda i: (i, 0)),
            pl.BlockSpec((1, _M), lambda i: (0, 0)),
        ],
        out_shape=[
            jax.ShapeDtypeStruct((_N, _L, enc_in), jnp.float32),
            jax.ShapeDtypeStruct((_L, _N, _D), jnp.float32),
            jax.ShapeDtypeStruct((_N, _L), jnp.float32),
            jax.ShapeDtypeStruct((1, _M), jnp.float32),
        ],
        scratch_shapes=[
            pltpu.VMEM((enc_in, enc_in), jnp.float32),
            pltpu.VMEM((enc_in, enc_in), jnp.float32),
            pltpu.VMEM((_M, enc_in), jnp.float32),
            pltpu.VMEM((1, enc_in), jnp.float32),
            pltpu.VMEM((_D, _M), jnp.float32),
            pltpu.VMEM((_M, 1), jnp.float32),
            pltpu.VMEM((_M, 1), jnp.float32),
        ],
        compiler_params=pltpu.CompilerParams(
            dimension_semantics=("arbitrary",),
        ),
    )(x, av, W_in, b_in2, W_tr, b_tr2, mem, wsig_row, bsig, wdec_t, b_dec2)

    season = seasonL.transpose(1, 0, 2)
    return (out3, mem, season, losses[0, 0], losses[0, 1], losses[0, 2],
            sigma2)


# speedup vs baseline: 15.0304x; 1.1493x over previous
import functools

import jax
import jax.numpy as jnp
import numpy as np
from jax.experimental import pallas as pl
from jax.experimental.pallas import tpu as pltpu

_D = 512
_M = 128
_L = 100
_N = 256
_TOPK = 8
_KWIN = 25
_NB = 32
_TOK = _NB * _L
_T = _N * _L


def _avg_matrix():
    pad = (_KWIN - 1) // 2
    src = np.clip(np.arange(_L + 2 * pad) - pad, 0, _L - 1)
    m = np.zeros((_L, _L), np.float32)
    for i in range(_L):
        for j in src[i:i + _KWIN]:
            m[i, j] += 1.0 / _KWIN
    return m


def _fused_body(x_ref, av_ref, w_in_ref, b_in_ref, w_tr_ref, b_tr_ref,
                mem_ref, wsig_ref, bsig_ref, w_dect_ref, b_dec_ref,
                out_ref, season_ref, sigma_ref, loss_ref,
                a1_ref, a2_ref, mw_ref, bias_ref, mem_t_ref, invn_ref,
                msq_ref):
    f32 = jnp.float32
    dot = functools.partial(jnp.dot, preferred_element_type=f32)

    def dot_t(a, b):
        return jax.lax.dot_general(a, b, (((1,), (1,)), ((), ())),
                                   preferred_element_type=f32)

    xb3 = x_ref[...]
    xb = xb3.reshape(_TOK, -1)
    w_in = w_in_ref[...]
    ones_d = jnp.full((_D, 1), 1.0, f32)

    @pl.when(pl.program_id(0) == 0)
    def _():
        wdt = w_dect_ref[...]
        w_tr = w_tr_ref[...]
        mem = mem_ref[...]
        a1_ref[...] = dot_t(w_in, wdt[:, 0:_D])
        a2_ref[...] = dot_t(dot(w_in, w_tr), wdt[:, _D:2 * _D])
        mw_ref[...] = dot_t(mem, wdt[:, 2 * _D:3 * _D])
        bias_ref[...] = (dot_t(b_in_ref[...], wdt[:, 0:_D])
                         + dot_t(dot(b_in_ref[...], w_tr) + b_tr_ref[...],
                                 wdt[:, _D:2 * _D])
                         + b_dec_ref[...])
        mem_t_ref[...] = jnp.transpose(mem)
        mnsq = dot(mem * mem, ones_d)
        invn_ref[...] = 10.0 / (jnp.sqrt(mnsq) + 1e-12)
        msq_ref[...] = mnsq
        loss_ref[...] = jnp.zeros_like(loss_ref)

    av = av_ref[...]
    xavg3 = jnp.stack([dot(av, xb3[n]) for n in range(_NB)], axis=0)
    xavg = xavg3.reshape(_TOK, -1)
    season = dot(xb - xavg, w_in)

    nsq = dot(season * season, ones_d)
    norm = jnp.sqrt(nsq)
    rnorm = 1.0 / (norm + 1e-12)
    q = season * rnorm
    for n in range(_NB):
        season_ref[:, n, :] = q[n * _L:(n + 1) * _L, :]

    att_raw = dot(q, mem_t_ref[...])
    art = jnp.transpose(att_raw)
    att = art * 10.0

    row_i = jax.lax.broadcasted_iota(jnp.int32, (_M, _M), 0)
    col_i = jax.lax.broadcasted_iota(jnp.int32, (_M, _M), 1)
    ltri = jnp.where(col_i < row_i, 1.0, 0.0).astype(f32)

    cur = att
    topmask = jnp.zeros((_M, _TOK), dtype=jnp.bool_)
    amax = jnp.max(att, axis=0, keepdims=True)
    mval = amax
    for it in range(_TOPK):
        eqm = cur == mval
        eqf = jnp.where(eqm, 1.0, 0.0)
        pc = dot(ltri, eqf)
        sel = jnp.logical_and(eqm, pc < 0.5)
        topmask = jnp.logical_or(topmask, sel)
        cur = jnp.where(sel, -jnp.inf, cur)
        if it + 1 < _TOPK:
            mval = jnp.max(cur, axis=0, keepdims=True)

    pt = jnp.where(topmask, jnp.exp(att - amax), 0.0)
    pt = pt / jnp.sum(pt, axis=0, keepdims=True)
    p = jnp.transpose(pt)

    out = dot(xb, a1_ref[...]) + dot(xavg, a2_ref[...]) + dot(p, mw_ref[...])
    out_ref[...] = (out + bias_ref[...]).reshape(_NB, _L, -1)

    sig = dot(q * wsig_ref[...], ones_d) + bsig_ref[0, 0]
    sigma_ref[...] = sig.reshape(_NB, _L)

    sim = art * invn_ref[...]
    smax = jnp.max(sim, axis=0, keepdims=True)
    lse = jnp.log(jnp.sum(jnp.exp(sim - smax), axis=0, keepdims=True)) + smax
    closs = jnp.sum(lse - smax) * (1.0 / _T)

    eqs = sim == smax
    eqsf = jnp.where(eqs, 1.0, 0.0)
    lsel = jnp.logical_and(eqs, dot(ltri, eqsf) < 0.5)
    att_sel = jnp.sum(jnp.where(lsel, art, 0.0))
    msq_sel = jnp.sum(jnp.where(lsel, msq_ref[...], 0.0))
    qsq = jnp.sum(nsq * rnorm * rnorm)
    gloss = (qsq - 2.0 * att_sel + msq_sel) * (1.0 / (_T * _D))

    sig2 = sig * sig + 1e-6
    kld = jnp.sum(sig2 - jnp.log(sig2) - 1.0) * (0.5 / _T)

    li = jax.lax.broadcasted_iota(jnp.int32, (1, _M), 1)
    vals = (jnp.where(li == 0, closs, 0.0) + jnp.where(li == 1, gloss, 0.0)
            + jnp.where(li == 2, kld, 0.0))
    loss_ref[...] += vals


@functools.partial(jax.jit)
def kernel(x, W_in, b_in, W_tr, b_tr, mem, W_sig, b_sig, W_dec, b_dec):
    enc_in = x.shape[2]
    av = jnp.asarray(_avg_matrix())
    wsig_row = W_sig.T
    wdec_t = W_dec.T
    bsig = b_sig.reshape(1, 1)
    b_in2 = b_in.reshape(1, _D)
    b_tr2 = b_tr.reshape(1, _D)
    b_dec2 = b_dec.reshape(1, -1)

    grid = (_T // _TOK,)
    tok3 = lambda cols: pl.BlockSpec((_NB, _L, cols), lambda i: (i, 0, 0))
    full = lambda shape: pl.BlockSpec(shape, lambda i: (0,) * len(shape))

    out3, seasonL, sigma2, losses = pl.pallas_call(
        _fused_body,
        grid=grid,
        in_specs=[
            tok3(enc_in),
            full((_L, _L)),
            full((enc_in, _D)),
            full((1, _D)),
            full((_D, _D)),
            full((1, _D)),
            full((_M, _D)),
            full((1, _D)),
            full((1, 1)),
            full((enc_in, 3 * _D)),
            full((1, enc_in)),
        ],
        out_specs=[
            tok3(enc_in),
            pl.BlockSpec((_L, _NB, _D), lambda i: (0, i, 0)),
            pl.BlockSpec((_NB, _L), lambda i: (i, 0)),
            pl.BlockSpec((1, _M), lambda i: (0, 0)),
        ],
        out_shape=[
            jax.ShapeDtypeStruct((_N, _L, enc_in), jnp.float32),
            jax.ShapeDtypeStruct((_L, _N, _D), jnp.float32),
            jax.ShapeDtypeStruct((_N, _L), jnp.float32),
            jax.ShapeDtypeStruct((1, _M), jnp.float32),
        ],
        scratch_shapes=[
            pltpu.VMEM((enc_in, enc_in), jnp.float32),
            pltpu.VMEM((enc_in, enc_in), jnp.float32),
            pltpu.VMEM((_M, enc_in), jnp.float32),
            pltpu.VMEM((1, enc_in), jnp.float32),
            pltpu.VMEM((_D, _M), jnp.float32),
            pltpu.VMEM((_M, 1), jnp.float32),
            pltpu.VMEM((_M, 1), jnp.float32),
        ],
        compiler_params=pltpu.CompilerParams(
            dimension_semantics=("arbitrary",),
        ),
    )(x, av, W_in, b_in2, W_tr, b_tr2, mem, wsig_row, bsig, wdec_t, b_dec2)

    season = seasonL.transpose(1, 0, 2)
    return (out3, mem, season, losses[0, 0], losses[0, 1], losses[0, 2],
            sigma2)
